# Initial kernel scaffold; baseline (speedup 1.0000x reference)
import jax
import jax.numpy as jnp
from jax.experimental import pallas as pl


def kernel(cost_matrix):
    raise NotImplementedError("write your pallas kernel here")


# speedup vs baseline: 1.8577x; 1.8577x over previous
import jax
import jax.numpy as jnp
from jax.experimental import pallas as pl

EMBED = 256
BATCH_BLOCK = 8


def _fill_body(row_ref, col_ref):
    row_ref[...] = jnp.zeros(row_ref.shape, jnp.float32)
    n = col_ref.shape[1]
    i = jax.lax.broadcasted_iota(jnp.int32, (n, EMBED), 0)
    j = jax.lax.broadcasted_iota(jnp.int32, (n, EMBED), 1)
    eye = (i == j).astype(jnp.float32)
    col_ref[...] = jnp.broadcast_to(eye[None], col_ref.shape)


def kernel(cost_matrix):
    b, r, c = cost_matrix.shape
    grid = (b // BATCH_BLOCK,)
    row_emb, col_emb = pl.pallas_call(
        _fill_body,
        grid=grid,
        out_specs=[
            pl.BlockSpec((BATCH_BLOCK, r, EMBED), lambda i: (i, 0, 0)),
            pl.BlockSpec((BATCH_BLOCK, c, EMBED), lambda i: (i, 0, 0)),
        ],
        out_shape=[
            jax.ShapeDtypeStruct((b, r, EMBED), cost_matrix.dtype),
            jax.ShapeDtypeStruct((b, c, EMBED), cost_matrix.dtype),
        ],
    )()
    return (row_emb, col_emb, cost_matrix)


# speedup vs baseline: 1.8706x; 1.0069x over previous
import jax
import jax.numpy as jnp
from jax.experimental import pallas as pl

EMBED = 256
BATCH_BLOCK = 32


def _fill_body(row_ref, col_ref):
    row_ref[...] = jnp.zeros(row_ref.shape, jnp.float32)
    n = col_ref.shape[1]
    i = jax.lax.broadcasted_iota(jnp.int32, (n, EMBED), 0)
    j = jax.lax.broadcasted_iota(jnp.int32, (n, EMBED), 1)
    eye = (i == j).astype(jnp.float32)
    col_ref[...] = jnp.broadcast_to(eye[None], col_ref.shape)


def kernel(cost_matrix):
    b, r, c = cost_matrix.shape
    grid = (b // BATCH_BLOCK,)
    row_emb, col_emb = pl.pallas_call(
        _fill_body,
        grid=grid,
        out_specs=[
            pl.BlockSpec((BATCH_BLOCK, r, EMBED), lambda i: (i, 0, 0)),
            pl.BlockSpec((BATCH_BLOCK, c, EMBED), lambda i: (i, 0, 0)),
        ],
        out_shape=[
            jax.ShapeDtypeStruct((b, r, EMBED), cost_matrix.dtype),
            jax.ShapeDtypeStruct((b, c, EMBED), cost_matrix.dtype),
        ],
    )()
    return (row_emb, col_emb, cost_matrix)
